# baseline (speedup 1.0000x reference)
import functools

import jax
import jax.numpy as jnp
from jax import lax
from jax.experimental import pallas as pl
from jax.experimental.pallas import tpu as pltpu
from jax.experimental.pallas import tpu_sc as plsc

_INFO = plsc.get_sparse_core_info()
_NC, _NS = _INFO.num_cores, _INFO.num_subcores
_NW = _NC * _NS
_L = 16


@functools.lru_cache(maxsize=None)
def _make_gather(b, h, v, d):
    assert b % _NW == 0
    bw = b // _NW
    mesh = plsc.VectorSubcoreMesh(core_axis_name="c", subcore_axis_name="s")

    @functools.partial(
        pl.kernel,
        out_type=jax.ShapeDtypeStruct((h, d, b), jnp.float32),
        mesh=mesh,
        scratch_types=[
            pltpu.VMEM((h, bw), jnp.int32),
            pltpu.VMEM((2, bw, d), jnp.float32),
            pltpu.VMEM((2, d, bw), jnp.float32),
            pltpu.SemaphoreType.DMA,
            pltpu.SemaphoreType.DMA,
        ],
        compiler_params=pltpu.CompilerParams(
            use_tc_tiling_on_sc=False, needs_layout_passes=False),
    )
    def gather_kernel(idx_t_hbm, table_hbm, out_t_hbm, idx_v, rows_v, trans_v,
                      gsem, osem):
        wid = lax.axis_index("s") * _NC + lax.axis_index("c")
        b0 = wid * bw
        pltpu.sync_copy(idx_t_hbm.at[:, pl.ds(b0, bw)], idx_v)

        def gather(j, t):
            pltpu.async_copy(table_hbm.at[idx_v.at[j]], rows_v.at[t], gsem)

        def store(j, t):
            pltpu.async_copy(trans_v.at[t], out_t_hbm.at[j, :, pl.ds(b0, bw)],
                             osem)

        def drain(sem):
            pltpu.make_async_copy(
                out_t_hbm.at[0, :, pl.ds(b0, bw)], trans_v.at[0], sem).wait()

        def transpose(t):
            @pl.loop(0, d)
            def _(dd):
                col = jnp.full((_L,), dd, jnp.int32)

                @pl.loop(0, bw, step=_L, unroll=4)
                def _(k0):
                    vec = plsc.load_gather(
                        rows_v.at[t],
                        [k0 + lax.iota(jnp.int32, _L), col])
                    trans_v[t, dd, pl.ds(k0, _L)] = vec

        gather(0, 0)

        @pl.loop(0, h, step=2)
        def _(j0):
            for t in (0, 1):
                j = j0 + t

                @pl.when(j + 1 < h)
                def _():
                    gather(j + 1, 1 - t)

                drain(gsem)

                @pl.when(j >= 2)
                def _():
                    drain(osem)

                transpose(t)
                store(j, t)

        drain(osem)
        drain(osem)

    return gather_kernel


def kernel(inputs, table):
    b, h = inputs.shape
    v, d = table.shape
    idx_t = inputs.astype(jnp.int32).T
    out_t = _make_gather(b, h, v, d)(idx_t, table)
    return out_t.transpose(2, 0, 1)


# speedup vs baseline: 1.1175x; 1.1175x over previous
import functools

import jax
import jax.numpy as jnp
from jax import lax
from jax.experimental import pallas as pl
from jax.experimental.pallas import tpu as pltpu
from jax.experimental.pallas import tpu_sc as plsc

_INFO = plsc.get_sparse_core_info()
_NC, _NS = _INFO.num_cores, _INFO.num_subcores
_NW = _NC * _NS
_L = 16


@functools.lru_cache(maxsize=None)
def _make_gather(b, h, v, d):
    assert b % _NW == 0
    bw = b // _NW
    mesh = plsc.VectorSubcoreMesh(core_axis_name="c", subcore_axis_name="s")

    @functools.partial(
        pl.kernel,
        out_type=jax.ShapeDtypeStruct((h, d, b), jnp.float32),
        mesh=mesh,
        scratch_types=[
            pltpu.VMEM((h, bw), jnp.int32),
            pltpu.VMEM((2, bw, d), jnp.float32),
            pltpu.VMEM((2, d, bw), jnp.float32),
            pltpu.SemaphoreType.DMA,
            pltpu.SemaphoreType.DMA,
        ],
        compiler_params=pltpu.CompilerParams(
            use_tc_tiling_on_sc=False, needs_layout_passes=False),
    )
    def gather_kernel(idx_t_hbm, table_hbm, out_t_hbm, idx_v, rows_v, trans_v,
                      gsem, osem):
        wid = lax.axis_index("s") * _NC + lax.axis_index("c")
        b0 = wid * bw
        pltpu.sync_copy(idx_t_hbm.at[:, pl.ds(b0, bw)], idx_v)
        iota = lax.iota(jnp.int32, _L)

        def gather(j, t):
            pltpu.async_copy(table_hbm.at[idx_v.at[j]], rows_v.at[t], gsem)

        def store(j, t):
            pltpu.async_copy(trans_v.at[t], out_t_hbm.at[j, :, pl.ds(b0, bw)],
                             osem)

        def drain(sem):
            pltpu.make_async_copy(
                out_t_hbm.at[0, :, pl.ds(b0, bw)], trans_v.at[0], sem).wait()

        def transpose(t):
            @pl.loop(0, d)
            def _(dd):
                col = jnp.full((_L,), dd, jnp.int32)

                @plsc.parallel_loop(0, bw, step=_L, unroll=8)
                def _(k0):
                    vec = plsc.load_gather(
                        rows_v.at[t], [k0 + iota, col])
                    trans_v[t, dd, pl.ds(k0, _L)] = vec

        gather(0, 0)

        @pl.loop(0, h, step=2)
        def _(j0):
            for t in (0, 1):
                j = j0 + t

                @pl.when(j + 1 < h)
                def _():
                    gather(j + 1, 1 - t)

                drain(gsem)

                @pl.when(j >= 2)
                def _():
                    drain(osem)

                transpose(t)
                store(j, t)

        drain(osem)
        drain(osem)

    return gather_kernel


def kernel(inputs, table):
    b, h = inputs.shape
    v, d = table.shape
    idx_t = inputs.astype(jnp.int32).T
    out_t = _make_gather(b, h, v, d)(idx_t, table)
    return out_t.transpose(2, 0, 1)


# speedup vs baseline: 1.2783x; 1.1438x over previous
import functools

import jax
import jax.numpy as jnp
from jax import lax
from jax.experimental import pallas as pl
from jax.experimental.pallas import tpu as pltpu
from jax.experimental.pallas import tpu_sc as plsc

_INFO = plsc.get_sparse_core_info()
_NC, _NS = _INFO.num_cores, _INFO.num_subcores
_NW = _NC * _NS


@functools.lru_cache(maxsize=None)
def _make_gather(b, h, v, d, nbuf):
    assert b % _NW == 0
    bw = b // _NW
    mesh = plsc.VectorSubcoreMesh(core_axis_name="c", subcore_axis_name="s")

    @functools.partial(
        pl.kernel,
        out_type=jax.ShapeDtypeStruct((h, b, d), jnp.float32),
        mesh=mesh,
        scratch_types=[
            pltpu.VMEM((h, bw), jnp.int32),
            pltpu.VMEM((nbuf, bw, d), jnp.float32),
            pltpu.SemaphoreType.DMA,
            pltpu.SemaphoreType.DMA,
        ],
        compiler_params=pltpu.CompilerParams(use_tc_tiling_on_sc=False),
    )
    def gather_kernel(idx_t_hbm, table_hbm, out_t_hbm, idx_v, rows_v, gsem, osem):
        wid = lax.axis_index("s") * _NC + lax.axis_index("c")
        b0 = wid * bw
        pltpu.sync_copy(idx_t_hbm.at[:, pl.ds(b0, bw)], idx_v)

        def do_chunk(j, t):
            g = pltpu.async_copy(table_hbm.at[idx_v.at[j]], rows_v.at[t], gsem)
            g.wait()
            pltpu.async_copy(rows_v.at[t], out_t_hbm.at[j, pl.ds(b0, bw)], osem)

        def drain_store():
            pltpu.make_async_copy(
                rows_v.at[0], out_t_hbm.at[0, pl.ds(b0, bw)], osem).wait()

        do_chunk(0, 0)
        do_chunk(1, 1)

        @pl.loop(2, h, step=nbuf)
        def _(j0):
            for t in range(nbuf):
                drain_store()
                do_chunk(j0 + t, t)

        for _t in range(nbuf):
            drain_store()

    return gather_kernel


def kernel(inputs, table):
    b, h = inputs.shape
    v, d = table.shape
    idx_t = inputs.astype(jnp.int32).T
    out_t = _make_gather(b, h, v, d, 2)(idx_t, table)
    return out_t.transpose(1, 0, 2)
